# baseline (speedup 1.0000x reference)
import functools

import jax
import jax.numpy as jnp
from jax import lax
from jax.experimental import pallas as pl
from jax.experimental.pallas import tpu as pltpu
from jax.experimental.pallas import tpu_sc as plsc

K = 32
CHUNK = 32
_NC, _NS, _L = 2, 16, 16
_NW = _NC * _NS


def _tc_body(a_ref, b_ref, u_ref, cm_ref, *, tm, tn):
    a = a_ref[...]
    b = b_ref[...]
    s = jax.lax.dot_general(
        a, b, dimension_numbers=(((1,), (1,)), ((), ())),
        preferred_element_type=jnp.float32)
    bn = jnp.sqrt(jnp.sum(b * b, axis=1))
    an = jnp.sqrt(jnp.sum(a * a, axis=1, keepdims=True))
    u = jnp.abs(s / (an * bn[None, :]))
    u_ref[...] = u
    r = u
    for sh in (1, 2, 4, 8, 16):
        r = jnp.maximum(r, jnp.concatenate([r[:, sh:], r[:, :sh]], axis=1))
    cols = jax.lax.broadcasted_iota(jnp.int32, (tn, tn // CHUNK), 0)
    sel = jax.lax.broadcasted_iota(jnp.int32, (tn, tn // CHUNK), 1)
    smat = (cols == sel * CHUNK).astype(jnp.float32)
    cm_ref[0, :, :] = jax.lax.dot_general(
        r, smat, dimension_numbers=(((1,), (0,)), ((), ())),
        precision=jax.lax.Precision.HIGHEST,
        preferred_element_type=jnp.float32)


def _tc_stage(x_actuators, x_sensors, tm=512, tn=2048):
    m, kd = x_actuators.shape
    n, _ = x_sensors.shape
    grid = (n // tn, m // tm)
    u, cm3 = pl.pallas_call(
        functools.partial(_tc_body, tm=tm, tn=tn),
        grid=grid,
        in_specs=[
            pl.BlockSpec((tm, kd), lambda j, i: (i, 0)),
            pl.BlockSpec((tn, kd), lambda j, i: (j, 0)),
        ],
        out_specs=[
            pl.BlockSpec((tm, tn), lambda j, i: (i, j)),
            pl.BlockSpec((1, tm, tn // CHUNK), lambda j, i: (j, i, 0)),
        ],
        out_shape=[
            jax.ShapeDtypeStruct((m, n), jnp.float32),
            jax.ShapeDtypeStruct((n // tn, m, tn // CHUNK), jnp.float32),
        ],
    )(x_actuators, x_sensors)
    cm = jnp.transpose(cm3, (1, 0, 2)).reshape(m, n // CHUNK)
    return u, cm


def _rev(x):
    return jax.lax.rev(x, (0,))


def _merge(av, ai, bv, bi, need_lo):
    brv, bri = _rev(bv), _rev(bi)
    m = av >= brv
    hv = jnp.where(m, av, brv)
    hi = jnp.where(m, ai, bri)
    hv, hi = plsc.sort_key_val(hv, hi, descending=True)
    if not need_lo:
        return hv, hi, None, None
    lv = jnp.where(m, brv, av)
    li = jnp.where(m, bri, ai)
    lv, li = plsc.sort_key_val(lv, li, descending=True)
    return hv, hi, lv, li


def _insert(r0v, r0i, r1v, r1i, xv, xi):
    xv, xi = plsc.sort_key_val(xv, xi, descending=True)
    r0v, r0i, restv, resti = _merge(r0v, r0i, xv, xi, True)
    r1v, r1i, _, _ = _merge(restv, resti, r1v, r1i, False)
    return r0v, r0i, r1v, r1i


_SPLAT15 = None


def _maybe_insert(r0v, r0i, r1v, r1i, xv, xi):
    thr = r1v[jnp.full((_L,), 15, jnp.int32)]
    hit = jnp.any(xv > thr)
    return lax.cond(
        hit,
        lambda ops: _insert(*ops),
        lambda ops: ops[:4],
        (r0v, r0i, r1v, r1i, xv, xi),
    )


def _sc_body(u2, cm, vals, idx, cm_v, gath_v, gidx_v, wv_v, wi_v,
             outv_v, outi_v, sem, *, m, nchunks, rpw):
    wid = lax.axis_index("c") * _NS + lax.axis_index("s")
    base = wid * rpw
    pltpu.sync_copy(cm.at[pl.ds(base, rpw)], cm_v)

    iota = lax.iota(jnp.int32, _L)
    neg1 = jnp.full((_L,), -1.0, jnp.float32)
    zeroi = jnp.zeros((_L,), jnp.int32)

    def row_body(rl, carry):
        r0v, r1v = neg1, neg1
        r0i, r1i = zeroi, zeroi
        for v in range(nchunks // _L):
            x = cm_v[rl, pl.ds(v * _L, _L)]
            ci = iota + (v * _L)
            if v < 2:
                r0v, r0i, r1v, r1i = _insert(r0v, r0i, r1v, r1i, x, ci)
            else:
                r0v, r0i, r1v, r1i = _maybe_insert(r0v, r0i, r1v, r1i, x, ci)
        rowbase = (base + rl) * (nchunks // 4)
        gidx_v[pl.ds(0, _L)] = (r0i >> 2) + rowbase
        gidx_v[pl.ds(_L, _L)] = (r1i >> 2) + rowbase
        wv_v[pl.ds(0, _L)] = r0v
        wv_v[pl.ds(_L, _L)] = r1v
        wi_v[pl.ds(0, _L)] = r0i
        wi_v[pl.ds(_L, _L)] = r1i
        pltpu.async_copy(u2.at[gidx_v], gath_v, sem).wait()
        def sb_cond(carry):
            s, r0v, r0i, r1v, r1i = carry
            thr = r1v[jnp.full((_L,), 15, jnp.int32)]
            cmax = plsc.load_gather(wv_v, [jnp.full((_L,), 0, jnp.int32) + s])
            return jnp.logical_and(s < K, jnp.any(cmax > thr))

        def sb_body(carry):
            s, r0v, r0i, r1v, r1i = carry
            splat = jnp.full((_L,), 0, jnp.int32) + s
            cid = plsc.load_gather(wi_v, [splat])
            seg = (cid & 3) * CHUNK
            for h in range(CHUNK // _L):
                lane = seg + (h * _L) + iota
                x = plsc.load_gather(gath_v, [splat, lane])
                col = cid * CHUNK + (h * _L) + iota
                r0v, r0i, r1v, r1i = _insert(r0v, r0i, r1v, r1i, x, col)
            return s + 1, r0v, r0i, r1v, r1i

        _, r0v, r0i, r1v, r1i = lax.while_loop(
            sb_cond, sb_body, (jnp.int32(0), neg1, zeroi, neg1, zeroi))
        outv_v[rl, pl.ds(0, _L)] = r0v
        outv_v[rl, pl.ds(_L, _L)] = r1v
        outi_v[rl, pl.ds(0, _L)] = r0i
        outi_v[rl, pl.ds(_L, _L)] = r1i
        return carry

    jax.lax.fori_loop(0, rpw, row_body, 0)
    pltpu.sync_copy(outv_v, vals.at[pl.ds(base, rpw)])
    pltpu.sync_copy(outi_v, idx.at[pl.ds(base, rpw)])


def _sc_topk(u, cm):
    m, n = u.shape
    nchunks = n // CHUNK
    rpw = m // _NW
    u2 = u.reshape(m * (n // 128), 128)
    mesh = plsc.VectorSubcoreMesh(core_axis_name="c", subcore_axis_name="s")
    fn = pl.kernel(
        functools.partial(_sc_body, m=m, nchunks=nchunks, rpw=rpw),
        out_type=[
            jax.ShapeDtypeStruct((m, K), jnp.float32),
            jax.ShapeDtypeStruct((m, K), jnp.int32),
        ],
        mesh=mesh,
        compiler_params=pltpu.CompilerParams(needs_layout_passes=False),
        scratch_types=[
            pltpu.VMEM((rpw, nchunks), jnp.float32),
            pltpu.VMEM((K, 128), jnp.float32),
            pltpu.VMEM((K,), jnp.int32),
            pltpu.VMEM((K,), jnp.float32),
            pltpu.VMEM((K,), jnp.int32),
            pltpu.VMEM((rpw, K), jnp.float32),
            pltpu.VMEM((rpw, K), jnp.int32),
            pltpu.SemaphoreType.DMA,
        ],
    )
    return fn(u2, cm)


def kernel(x_actuators, x_sensors):
    m = x_actuators.shape[0]
    nh = 2
    mh = m // nh
    parts = []
    for h in range(nh):
        u, cm = _tc_stage(x_actuators[h * mh:(h + 1) * mh], x_sensors)
        parts.append(_sc_topk(u, cm))
    vals = jnp.concatenate([p[0] for p in parts], axis=0)
    idx = jnp.concatenate([p[1] for p in parts], axis=0)
    weights = vals.reshape(-1)
    source = jnp.repeat(jnp.arange(m, dtype=jnp.int32), K)
    edges = jnp.stack([source, idx.reshape(-1)], axis=0)
    return (edges, weights)


# speedup vs baseline: 1.1633x; 1.1633x over previous
import functools

import jax
import jax.numpy as jnp
from jax import lax
from jax.experimental import pallas as pl
from jax.experimental.pallas import tpu as pltpu
from jax.experimental.pallas import tpu_sc as plsc

K = 32
CHUNK = 32
_NC, _NS, _L = 2, 16, 16
_NW = _NC * _NS


def _tc_body(a_ref, b_ref, u_ref, cm_ref, *, tm, tn):
    a = a_ref[...]
    b = b_ref[...]
    s = jax.lax.dot_general(
        a, b, dimension_numbers=(((1,), (1,)), ((), ())),
        preferred_element_type=jnp.float32)
    bn = jnp.sqrt(jnp.sum(b * b, axis=1))
    an = jnp.sqrt(jnp.sum(a * a, axis=1, keepdims=True))
    u = jnp.abs(s / (an * bn[None, :]))
    u_ref[...] = u
    r = u
    for sh in (1, 2, 4, 8, 16):
        r = jnp.maximum(r, jnp.concatenate([r[:, sh:], r[:, :sh]], axis=1))
    cols = jax.lax.broadcasted_iota(jnp.int32, (tn, tn // CHUNK), 0)
    sel = jax.lax.broadcasted_iota(jnp.int32, (tn, tn // CHUNK), 1)
    smat = (cols == sel * CHUNK).astype(jnp.float32)
    cm_ref[0, :, :] = jax.lax.dot_general(
        r, smat, dimension_numbers=(((1,), (0,)), ((), ())),
        precision=jax.lax.Precision.HIGHEST,
        preferred_element_type=jnp.float32)


def _tc_stage(x_actuators, x_sensors, tm=512, tn=2048):
    m, kd = x_actuators.shape
    n, _ = x_sensors.shape
    grid = (n // tn, m // tm)
    u, cm3 = pl.pallas_call(
        functools.partial(_tc_body, tm=tm, tn=tn),
        grid=grid,
        in_specs=[
            pl.BlockSpec((tm, kd), lambda j, i: (i, 0)),
            pl.BlockSpec((tn, kd), lambda j, i: (j, 0)),
        ],
        out_specs=[
            pl.BlockSpec((tm, tn), lambda j, i: (i, j)),
            pl.BlockSpec((1, tm, tn // CHUNK), lambda j, i: (j, i, 0)),
        ],
        out_shape=[
            jax.ShapeDtypeStruct((m, n), jnp.float32),
            jax.ShapeDtypeStruct((n // tn, m, tn // CHUNK), jnp.float32),
        ],
    )(x_actuators, x_sensors)
    cm = jnp.transpose(cm3, (1, 0, 2)).reshape(m, n // CHUNK)
    return u, cm


def _rev(x):
    return jax.lax.rev(x, (0,))


def _merge(av, ai, bv, bi, need_lo):
    brv, bri = _rev(bv), _rev(bi)
    m = av >= brv
    hv = jnp.where(m, av, brv)
    hi = jnp.where(m, ai, bri)
    hv, hi = plsc.sort_key_val(hv, hi, descending=True)
    if not need_lo:
        return hv, hi, None, None
    lv = jnp.where(m, brv, av)
    li = jnp.where(m, bri, ai)
    lv, li = plsc.sort_key_val(lv, li, descending=True)
    return hv, hi, lv, li


def _insert(r0v, r0i, r1v, r1i, xv, xi):
    xv, xi = plsc.sort_key_val(xv, xi, descending=True)
    r0v, r0i, restv, resti = _merge(r0v, r0i, xv, xi, True)
    r1v, r1i, _, _ = _merge(restv, resti, r1v, r1i, False)
    return r0v, r0i, r1v, r1i


_SPLAT15 = None


def _maybe_insert(r0v, r0i, r1v, r1i, xv, xi):
    thr = r1v[jnp.full((_L,), 15, jnp.int32)]
    hit = jnp.any(xv > thr)
    return lax.cond(
        hit,
        lambda ops: _insert(*ops),
        lambda ops: ops[:4],
        (r0v, r0i, r1v, r1i, xv, xi),
    )


def _sc_body(u2, cm, vals, idx, cm_v, gath_v, gidx_v, wv_v, wi_v,
             outv_v, outi_v, sem, *, m, nchunks, rpw):
    wid = lax.axis_index("c") * _NS + lax.axis_index("s")
    base = wid * rpw
    pltpu.sync_copy(cm.at[pl.ds(base, rpw)], cm_v)

    iota = lax.iota(jnp.int32, _L)
    neg1 = jnp.full((_L,), -1.0, jnp.float32)
    zeroi = jnp.zeros((_L,), jnp.int32)

    def row_body(rl, carry):
        r0v, r1v = neg1, neg1
        r0i, r1i = zeroi, zeroi
        for v in range(nchunks // _L):
            x = cm_v[rl, pl.ds(v * _L, _L)]
            ci = iota + (v * _L)
            if v < 2:
                r0v, r0i, r1v, r1i = _insert(r0v, r0i, r1v, r1i, x, ci)
            else:
                r0v, r0i, r1v, r1i = _maybe_insert(r0v, r0i, r1v, r1i, x, ci)
        rowbase = (base + rl) * (nchunks // 4)
        gidx_v[pl.ds(0, _L)] = (r0i >> 2) + rowbase
        gidx_v[pl.ds(_L, _L)] = (r1i >> 2) + rowbase
        wv_v[pl.ds(0, _L)] = r0v
        wv_v[pl.ds(_L, _L)] = r1v
        wi_v[pl.ds(0, _L)] = r0i
        wi_v[pl.ds(_L, _L)] = r1i
        pltpu.async_copy(u2.at[gidx_v], gath_v, sem).wait()
        def sb_cond(carry):
            s, r0v, r0i, r1v, r1i = carry
            thr = r1v[jnp.full((_L,), 15, jnp.int32)]
            cmax = plsc.load_gather(wv_v, [jnp.full((_L,), 0, jnp.int32) + s])
            return jnp.logical_and(s < K, jnp.any(cmax > thr))

        def sb_body(carry):
            s, r0v, r0i, r1v, r1i = carry
            splat = jnp.full((_L,), 0, jnp.int32) + s
            cid = plsc.load_gather(wi_v, [splat])
            seg = (cid & 3) * CHUNK
            for h in range(CHUNK // _L):
                lane = seg + (h * _L) + iota
                x = plsc.load_gather(gath_v, [splat, lane])
                col = cid * CHUNK + (h * _L) + iota
                r0v, r0i, r1v, r1i = _insert(r0v, r0i, r1v, r1i, x, col)
            return s + 1, r0v, r0i, r1v, r1i

        _, r0v, r0i, r1v, r1i = lax.while_loop(
            sb_cond, sb_body, (jnp.int32(0), neg1, zeroi, neg1, zeroi))
        outv_v[rl, pl.ds(0, _L)] = r0v
        outv_v[rl, pl.ds(_L, _L)] = r1v
        outi_v[rl, pl.ds(0, _L)] = r0i
        outi_v[rl, pl.ds(_L, _L)] = r1i
        return carry

    jax.lax.fori_loop(0, rpw, row_body, 0)
    pltpu.sync_copy(outv_v, vals.at[pl.ds(base, rpw)])
    pltpu.sync_copy(outi_v, idx.at[pl.ds(base, rpw)])


def _sc_topk(u, cm):
    m, n = u.shape
    nchunks = n // CHUNK
    rpw = m // _NW
    u2 = u.reshape(m * (n // 128), 128)
    mesh = plsc.VectorSubcoreMesh(core_axis_name="c", subcore_axis_name="s")
    fn = pl.kernel(
        functools.partial(_sc_body, m=m, nchunks=nchunks, rpw=rpw),
        out_type=[
            jax.ShapeDtypeStruct((m, K), jnp.float32),
            jax.ShapeDtypeStruct((m, K), jnp.int32),
        ],
        mesh=mesh,
        compiler_params=pltpu.CompilerParams(needs_layout_passes=False),
        scratch_types=[
            pltpu.VMEM((rpw, nchunks), jnp.float32),
            pltpu.VMEM((K, 128), jnp.float32),
            pltpu.VMEM((K,), jnp.int32),
            pltpu.VMEM((K,), jnp.float32),
            pltpu.VMEM((K,), jnp.int32),
            pltpu.VMEM((rpw, K), jnp.float32),
            pltpu.VMEM((rpw, K), jnp.int32),
            pltpu.SemaphoreType.DMA,
        ],
    )
    return fn(u2, cm)


def kernel(x_actuators, x_sensors):
    m = x_actuators.shape[0]
    nh = 8
    mh = m // nh
    parts = []
    for h in range(nh):
        u, cm = _tc_stage(x_actuators[h * mh:(h + 1) * mh], x_sensors)
        parts.append(_sc_topk(u, cm))
    vals = jnp.concatenate([p[0] for p in parts], axis=0)
    idx = jnp.concatenate([p[1] for p in parts], axis=0)
    weights = vals.reshape(-1)
    source = jnp.repeat(jnp.arange(m, dtype=jnp.int32), K)
    edges = jnp.stack([source, idx.reshape(-1)], axis=0)
    return (edges, weights)


# speedup vs baseline: 1.1640x; 1.0006x over previous
import functools

import jax
import jax.numpy as jnp
from jax import lax
from jax.experimental import pallas as pl
from jax.experimental.pallas import tpu as pltpu
from jax.experimental.pallas import tpu_sc as plsc

K = 32
CHUNK = 32
_NC, _NS, _L = 2, 16, 16
_NW = _NC * _NS


def _tc_body(a_ref, b_ref, u_ref, cm_ref, *, tm, tn):
    a = a_ref[...]
    b = b_ref[...]
    s = jax.lax.dot_general(
        a, b, dimension_numbers=(((1,), (1,)), ((), ())),
        preferred_element_type=jnp.float32)
    bn = jnp.sqrt(jnp.sum(b * b, axis=1))
    an = jnp.sqrt(jnp.sum(a * a, axis=1, keepdims=True))
    u = jnp.abs(s / (an * bn[None, :]))
    u_ref[...] = u
    r = u
    for sh in (1, 2, 4, 8, 16):
        r = jnp.maximum(r, jnp.concatenate([r[:, sh:], r[:, :sh]], axis=1))
    cols = jax.lax.broadcasted_iota(jnp.int32, (tn, tn // CHUNK), 0)
    sel = jax.lax.broadcasted_iota(jnp.int32, (tn, tn // CHUNK), 1)
    smat = (cols == sel * CHUNK).astype(jnp.float32)
    cm_ref[0, :, :] = jax.lax.dot_general(
        r, smat, dimension_numbers=(((1,), (0,)), ((), ())),
        precision=jax.lax.Precision.HIGHEST,
        preferred_element_type=jnp.float32)


def _tc_stage(x_actuators, x_sensors, tm=512, tn=2048):
    m, kd = x_actuators.shape
    n, _ = x_sensors.shape
    grid = (n // tn, m // tm)
    u, cm3 = pl.pallas_call(
        functools.partial(_tc_body, tm=tm, tn=tn),
        grid=grid,
        in_specs=[
            pl.BlockSpec((tm, kd), lambda j, i: (i, 0)),
            pl.BlockSpec((tn, kd), lambda j, i: (j, 0)),
        ],
        out_specs=[
            pl.BlockSpec((tm, tn), lambda j, i: (i, j)),
            pl.BlockSpec((1, tm, tn // CHUNK), lambda j, i: (j, i, 0)),
        ],
        out_shape=[
            jax.ShapeDtypeStruct((m, n), jnp.float32),
            jax.ShapeDtypeStruct((n // tn, m, tn // CHUNK), jnp.float32),
        ],
    )(x_actuators, x_sensors)
    cm = jnp.transpose(cm3, (1, 0, 2)).reshape(m, n // CHUNK)
    return u, cm


def _rev(x):
    return jax.lax.rev(x, (0,))


def _merge(av, ai, bv, bi, need_lo):
    brv, bri = _rev(bv), _rev(bi)
    m = av >= brv
    hv = jnp.where(m, av, brv)
    hi = jnp.where(m, ai, bri)
    hv, hi = plsc.sort_key_val(hv, hi, descending=True)
    if not need_lo:
        return hv, hi, None, None
    lv = jnp.where(m, brv, av)
    li = jnp.where(m, bri, ai)
    lv, li = plsc.sort_key_val(lv, li, descending=True)
    return hv, hi, lv, li


def _insert(r0v, r0i, r1v, r1i, xv, xi):
    xv, xi = plsc.sort_key_val(xv, xi, descending=True)
    r0v, r0i, restv, resti = _merge(r0v, r0i, xv, xi, True)
    r1v, r1i, _, _ = _merge(restv, resti, r1v, r1i, False)
    return r0v, r0i, r1v, r1i


def _maybe_insert(r0v, r0i, r1v, r1i, xv, xi):
    thr = r1v[jnp.full((_L,), 15, jnp.int32)]
    hit = jnp.any(xv > thr)
    return lax.cond(
        hit,
        lambda ops: _insert(*ops),
        lambda ops: ops[:4],
        (r0v, r0i, r1v, r1i, xv, xi),
    )


def _sc_body(u2, cm, vals, idx, cm_v, gath_v, gidx_v, wv_v, wi_v,
             outv_v, outi_v, sem, *, m, nchunks, rpw):
    wid = lax.axis_index("c") * _NS + lax.axis_index("s")
    base = wid * rpw
    pltpu.sync_copy(cm.at[pl.ds(base, rpw)], cm_v)

    iota = lax.iota(jnp.int32, _L)
    neg1 = jnp.full((_L,), -1.0, jnp.float32)
    zeroi = jnp.zeros((_L,), jnp.int32)

    def row_body(rl, carry):
        r0v, r1v = neg1, neg1
        r0i, r1i = zeroi, zeroi
        for v in range(nchunks // _L):
            x = cm_v[rl, pl.ds(v * _L, _L)]
            ci = iota + (v * _L)
            if v < 2:
                r0v, r0i, r1v, r1i = _insert(r0v, r0i, r1v, r1i, x, ci)
            else:
                r0v, r0i, r1v, r1i = _maybe_insert(r0v, r0i, r1v, r1i, x, ci)
        rowbase = (base + rl) * (nchunks // 4)
        gidx_v[pl.ds(0, _L)] = (r0i >> 2) + rowbase
        gidx_v[pl.ds(_L, _L)] = (r1i >> 2) + rowbase
        wv_v[pl.ds(0, _L)] = r0v
        wv_v[pl.ds(_L, _L)] = r1v
        wi_v[pl.ds(0, _L)] = r0i
        wi_v[pl.ds(_L, _L)] = r1i
        pltpu.async_copy(u2.at[gidx_v], gath_v, sem).wait()
        def sb_cond(carry):
            s, r0v, r0i, r1v, r1i = carry
            thr = r1v[jnp.full((_L,), 15, jnp.int32)]
            cmax = plsc.load_gather(wv_v, [jnp.full((_L,), 0, jnp.int32) + s])
            return jnp.logical_and(s < K, jnp.any(cmax > thr))

        def sb_body(carry):
            s, r0v, r0i, r1v, r1i = carry
            splat = jnp.full((_L,), 0, jnp.int32) + s
            cid = plsc.load_gather(wi_v, [splat])
            seg = (cid & 3) * CHUNK
            for h in range(CHUNK // _L):
                lane = seg + (h * _L) + iota
                x = plsc.load_gather(gath_v, [splat, lane])
                col = cid * CHUNK + (h * _L) + iota
                r0v, r0i, r1v, r1i = _insert(r0v, r0i, r1v, r1i, x, col)
            return s + 1, r0v, r0i, r1v, r1i

        _, r0v, r0i, r1v, r1i = lax.while_loop(
            sb_cond, sb_body, (jnp.int32(0), neg1, zeroi, neg1, zeroi))
        outv_v[rl, pl.ds(0, _L)] = r0v
        outv_v[rl, pl.ds(_L, _L)] = r1v
        outi_v[rl, pl.ds(0, _L)] = r0i
        outi_v[rl, pl.ds(_L, _L)] = r1i
        return carry

    jax.lax.fori_loop(0, rpw, row_body, 0)
    pltpu.sync_copy(outv_v, vals.at[pl.ds(base, rpw)])
    pltpu.sync_copy(outi_v, idx.at[pl.ds(base, rpw)])


def _sc_topk(u, cm):
    m, n = u.shape
    nchunks = n // CHUNK
    rpw = m // _NW
    u2 = u.reshape(m * (n // 128), 128)
    mesh = plsc.VectorSubcoreMesh(core_axis_name="c", subcore_axis_name="s")
    fn = pl.kernel(
        functools.partial(_sc_body, m=m, nchunks=nchunks, rpw=rpw),
        out_type=[
            jax.ShapeDtypeStruct((m, K), jnp.float32),
            jax.ShapeDtypeStruct((m, K), jnp.int32),
        ],
        mesh=mesh,
        compiler_params=pltpu.CompilerParams(needs_layout_passes=False),
        scratch_types=[
            pltpu.VMEM((rpw, nchunks), jnp.float32),
            pltpu.VMEM((K, 128), jnp.float32),
            pltpu.VMEM((K,), jnp.int32),
            pltpu.VMEM((K,), jnp.float32),
            pltpu.VMEM((K,), jnp.int32),
            pltpu.VMEM((rpw, K), jnp.float32),
            pltpu.VMEM((rpw, K), jnp.int32),
            pltpu.SemaphoreType.DMA,
        ],
    )
    return fn(u2, cm)


def kernel(x_actuators, x_sensors):
    m = x_actuators.shape[0]
    nh = 8
    mh = m // nh
    parts = []
    for h in range(nh):
        u, cm = _tc_stage(x_actuators[h * mh:(h + 1) * mh], x_sensors)
        parts.append(_sc_topk(u, cm))
    vals = jnp.concatenate([p[0] for p in parts], axis=0)
    idx = jnp.concatenate([p[1] for p in parts], axis=0)
    weights = vals.reshape(-1)
    source = jnp.repeat(jnp.arange(m, dtype=jnp.int32), K)
    edges = jnp.stack([source, idx.reshape(-1)], axis=0)
    return (edges, weights)
